# baseline (speedup 1.0000x reference)
import functools

import jax
import jax.numpy as jnp
from jax import lax
from jax.experimental import pallas as pl
from jax.experimental.pallas import tpu as pltpu
import jax.experimental.pallas.tpu_sc as plsc

N_NODES = 10000
N_EDGES = 320000
D_NODE = 128
D_EDGE = 16
D_OUT = 128

NC = 2
NS = 16
NW = NC * NS
EPW = N_EDGES // NW
CH = 40
NCHUNK = EPW // CH
N_PAD = 10240
ROWS_PT = N_PAD // NS
LANES = 16
GRP = D_OUT // LANES


def _g_body(x_ref, wx_ref, g_ref):
    g_ref[...] = lax.dot_general(
        x_ref[...], wx_ref[...], (((1,), (1,)), ((), ())),
        preferred_element_type=jnp.float32)


def _a_body(ea_ref, we_ref, b_ref, a_ref):
    a_ref[...] = lax.dot_general(
        ea_ref[...], we_ref[...], (((1,), (1,)), ((), ())),
        preferred_element_type=jnp.float32) + b_ref[...]


NBUF = 3


def _sc_body(g_hbm, src_hbm, dst_hbm, a_hbm, zero_hbm, out_hbm,
             sidx0, sidx1, sidx2, didx0, didx1, didx2,
             rows0, rows1, rows2, msg0, msg1, msg2, acc,
             sg0, sg1, sg2, sa0, sa1, sa2,
             sis0, sis1, sis2, sid0, sid1, sid2, ss0, ss1, ss2):
    sidx = (sidx0, sidx1, sidx2)
    didx = (didx0, didx1, didx2)
    rows = (rows0, rows1, rows2)
    msg = (msg0, msg1, msg2)
    sg = (sg0, sg1, sg2)
    sa = (sa0, sa1, sa2)
    sis = (sis0, sis1, sis2)
    sdi = (sid0, sid1, sid2)
    ss = (ss0, ss1, ss2)

    cid = lax.axis_index("c")
    sid_ax = lax.axis_index("s")
    wid = sid_ax * NC + cid
    ebase = wid * EPW

    def issue_idx_feeds(c, b):
        off = ebase + c * CH
        pltpu.async_copy(src_hbm.at[pl.ds(off, CH)], sidx[b], sis[b])
        pltpu.async_copy(dst_hbm.at[pl.ds(off, CH)], didx[b], sdi[b])
        pltpu.async_copy(a_hbm.at[pl.ds(off, CH)], msg[b], sa[b])

    def issue_gather(b):
        pltpu.make_async_copy(src_hbm.at[pl.ds(0, CH)], sidx[b], sis[b]).wait()
        pltpu.async_copy(g_hbm.at[sidx[b]], rows[b], sg[b])

    def wait_feeds(b):
        pltpu.make_async_copy(dst_hbm.at[pl.ds(0, CH)], didx[b], sdi[b]).wait()
        pltpu.make_async_copy(a_hbm.at[pl.ds(0, CH)], msg[b], sa[b]).wait()
        pltpu.make_async_copy(g_hbm.at[sidx[b]], rows[b], sg[b]).wait()

    def compute_scatter(b):
        def edge_body(e, c2):
            for j in range(GRP):
                s = pl.ds(j * LANES, LANES)
                msg[b][e, s] = jnp.maximum(rows[b][e, s] + msg[b][e, s], 0.0)
            return c2

        lax.fori_loop(0, CH, edge_body, 0, unroll=2)
        pltpu.async_copy(msg[b], acc.at[didx[b]], ss[b], add=True)

    def wait_scatter(b):
        pltpu.make_async_copy(msg[b], acc.at[didx[b]], ss[b]).wait()

    issue_idx_feeds(0, 0)
    issue_idx_feeds(1, 1)
    issue_gather(0)
    pltpu.sync_copy(zero_hbm.at[pl.ds(sid_ax * ROWS_PT, ROWS_PT)],
                    acc.at[pl.ds(sid_ax * ROWS_PT, ROWS_PT)])
    plsc.subcore_barrier()

    wait_feeds(0)
    compute_scatter(0)
    issue_idx_feeds(2, 2)
    issue_gather(1)

    wait_feeds(1)
    compute_scatter(1)
    wait_scatter(0)
    issue_idx_feeds(3, 0)
    issue_gather(2)

    @pl.loop(0, (NCHUNK - 4) // NBUF)
    def _grp(k):
        for b in range(NBUF):
            c = NBUF * k + 2 + b
            buf = (2 + b) % NBUF
            prv = (buf + 2) % NBUF
            nxt = (buf + 1) % NBUF
            wait_feeds(buf)
            compute_scatter(buf)
            wait_scatter(prv)
            issue_idx_feeds(c + 2, prv)
            issue_gather(nxt)

    c = NCHUNK - 2
    buf = c % NBUF
    wait_feeds(buf)
    compute_scatter(buf)
    wait_scatter((buf + 2) % NBUF)
    issue_gather((buf + 1) % NBUF)

    c = NCHUNK - 1
    buf = c % NBUF
    wait_feeds(buf)
    compute_scatter(buf)
    wait_scatter((buf + 2) % NBUF)
    wait_scatter(buf)

    plsc.subcore_barrier()
    pltpu.sync_copy(acc.at[pl.ds(sid_ax * ROWS_PT, ROWS_PT)],
                    out_hbm.at[cid, pl.ds(sid_ax * ROWS_PT, ROWS_PT)])


def _combine_body(p_ref, o_ref):
    o_ref[...] = p_ref[0] + p_ref[1]


def kernel(x, edge_index, edge_attr, W, b):
    src = edge_index[0].astype(jnp.int32)
    dst = edge_index[1].astype(jnp.int32)
    We = W[:, :D_EDGE]
    Wx = W[:, D_EDGE:]
    b2 = b.reshape(1, D_OUT)
    zeros = jnp.zeros((N_PAD, D_OUT), jnp.float32)

    g = pl.pallas_call(
        _g_body,
        out_shape=jax.ShapeDtypeStruct((N_NODES, D_NODE), jnp.float32),
        grid=(5,),
        in_specs=[
            pl.BlockSpec((N_NODES // 5, D_NODE), lambda i: (i, 0)),
            pl.BlockSpec((D_OUT, D_NODE), lambda i: (0, 0)),
        ],
        out_specs=pl.BlockSpec((N_NODES // 5, D_NODE), lambda i: (i, 0)),
    )(x, Wx)

    BLK_E = 6400
    a = pl.pallas_call(
        _a_body,
        out_shape=jax.ShapeDtypeStruct((N_EDGES, D_OUT), jnp.float32),
        grid=(N_EDGES // BLK_E,),
        in_specs=[
            pl.BlockSpec((BLK_E, D_EDGE), lambda i: (i, 0)),
            pl.BlockSpec((D_OUT, D_EDGE), lambda i: (0, 0)),
            pl.BlockSpec((1, D_OUT), lambda i: (0, 0)),
        ],
        out_specs=pl.BlockSpec((BLK_E, D_OUT), lambda i: (i, 0)),
    )(edge_attr, We, b2)

    sc_call = pl.kernel(
        _sc_body,
        out_type=jax.ShapeDtypeStruct((NC, N_PAD, D_OUT), jnp.float32),
        mesh=plsc.VectorSubcoreMesh(core_axis_name="c", subcore_axis_name="s"),
        scratch_types=(
            [pltpu.VMEM((CH,), jnp.int32) for _ in range(2 * NBUF)]
            + [pltpu.VMEM((CH, D_OUT), jnp.float32) for _ in range(2 * NBUF)]
            + [pltpu.VMEM_SHARED((N_PAD, D_OUT), jnp.float32)]
            + [pltpu.SemaphoreType.DMA for _ in range(5 * NBUF)]
        ),
    )
    partials = sc_call(g, src, dst, a, zeros)

    out = pl.pallas_call(
        _combine_body,
        out_shape=jax.ShapeDtypeStruct((N_PAD, D_OUT), jnp.float32),
        grid=(8,),
        in_specs=[pl.BlockSpec((NC, N_PAD // 8, D_OUT), lambda i: (0, i, 0))],
        out_specs=pl.BlockSpec((N_PAD // 8, D_OUT), lambda i: (i, 0)),
    )(partials)
    return out[:N_NODES]


# speedup vs baseline: 1.4069x; 1.4069x over previous
import functools

import jax
import jax.numpy as jnp
from jax import lax
from jax.experimental import pallas as pl
from jax.experimental.pallas import tpu as pltpu
import jax.experimental.pallas.tpu_sc as plsc

N_NODES = 10000
N_EDGES = 320000
D_NODE = 128
D_EDGE = 16
D_OUT = 128

NC = 2
NS = 16
NW = NC * NS
EPW = N_EDGES // NW
CH = 40
NCHUNK = EPW // CH
N_PAD = 10240
ROWS_PT = N_PAD // NS
LANES = 16
GRP = D_OUT // LANES


def _g_body(x_ref, wx_ref, g_ref):
    g_ref[...] = lax.dot_general(
        x_ref[...], wx_ref[...], (((1,), (1,)), ((), ())),
        preferred_element_type=jnp.float32)


def _a_body(ea_ref, we_ref, b_ref, a_ref):
    a_ref[...] = lax.dot_general(
        ea_ref[...], we_ref[...], (((1,), (1,)), ((), ())),
        preferred_element_type=jnp.float32) + b_ref[...]


NBUF = 3


def _sc_body(g_hbm, src_hbm, dst_hbm, a_hbm, zero_hbm, out_hbm,
             sidx0, sidx1, sidx2, didx0, didx1, didx2,
             rows0, rows1, rows2, msg0, msg1, msg2, acc,
             sg0, sg1, sg2, sa0, sa1, sa2,
             sis0, sis1, sis2, sid0, sid1, sid2, ss0, ss1, ss2):
    sidx = (sidx0, sidx1, sidx2)
    didx = (didx0, didx1, didx2)
    rows = (rows0, rows1, rows2)
    msg = (msg0, msg1, msg2)
    sg = (sg0, sg1, sg2)
    sa = (sa0, sa1, sa2)
    sis = (sis0, sis1, sis2)
    sdi = (sid0, sid1, sid2)
    ss = (ss0, ss1, ss2)

    cid = lax.axis_index("c")
    sid_ax = lax.axis_index("s")
    wid = sid_ax * NC + cid
    ebase = wid * EPW

    def issue_idx_feeds(c, b):
        off = ebase + c * CH
        pltpu.async_copy(src_hbm.at[pl.ds(off, CH)], sidx[b], sis[b])
        pltpu.async_copy(dst_hbm.at[pl.ds(off, CH)], didx[b], sdi[b])
        pltpu.async_copy(a_hbm.at[pl.ds(off, CH)], msg[b], sa[b])

    def issue_gather(b):
        pltpu.make_async_copy(src_hbm.at[pl.ds(0, CH)], sidx[b], sis[b]).wait()
        pltpu.async_copy(g_hbm.at[sidx[b]], rows[b], sg[b])

    def wait_feeds(b):
        pltpu.make_async_copy(dst_hbm.at[pl.ds(0, CH)], didx[b], sdi[b]).wait()
        pltpu.make_async_copy(a_hbm.at[pl.ds(0, CH)], msg[b], sa[b]).wait()
        pltpu.make_async_copy(g_hbm.at[sidx[b]], rows[b], sg[b]).wait()

    def compute_scatter(b):
        @plsc.parallel_loop(0, CH, 1, unroll=4)
        def edge_body(e):
            for j in range(GRP):
                s = pl.ds(j * LANES, LANES)
                msg[b][e, s] = jnp.maximum(rows[b][e, s] + msg[b][e, s], 0.0)

        pltpu.async_copy(msg[b], acc.at[didx[b]], ss[b], add=True)

    def wait_scatter(b):
        pltpu.make_async_copy(msg[b], acc.at[didx[b]], ss[b]).wait()

    issue_idx_feeds(0, 0)
    issue_idx_feeds(1, 1)
    issue_gather(0)
    pltpu.sync_copy(zero_hbm.at[pl.ds(sid_ax * ROWS_PT, ROWS_PT)],
                    acc.at[pl.ds(sid_ax * ROWS_PT, ROWS_PT)])
    plsc.subcore_barrier()

    wait_feeds(0)
    compute_scatter(0)
    issue_idx_feeds(2, 2)
    issue_gather(1)

    wait_feeds(1)
    compute_scatter(1)
    wait_scatter(0)
    issue_idx_feeds(3, 0)
    issue_gather(2)

    @pl.loop(0, (NCHUNK - 4) // NBUF)
    def _grp(k):
        for b in range(NBUF):
            c = NBUF * k + 2 + b
            buf = (2 + b) % NBUF
            prv = (buf + 2) % NBUF
            nxt = (buf + 1) % NBUF
            wait_feeds(buf)
            compute_scatter(buf)
            wait_scatter(prv)
            issue_idx_feeds(c + 2, prv)
            issue_gather(nxt)

    c = NCHUNK - 2
    buf = c % NBUF
    wait_feeds(buf)
    compute_scatter(buf)
    wait_scatter((buf + 2) % NBUF)
    issue_gather((buf + 1) % NBUF)

    c = NCHUNK - 1
    buf = c % NBUF
    wait_feeds(buf)
    compute_scatter(buf)
    wait_scatter((buf + 2) % NBUF)
    wait_scatter(buf)

    plsc.subcore_barrier()
    pltpu.sync_copy(acc.at[pl.ds(sid_ax * ROWS_PT, ROWS_PT)],
                    out_hbm.at[cid, pl.ds(sid_ax * ROWS_PT, ROWS_PT)])


def _combine_body(p_ref, o_ref):
    o_ref[...] = p_ref[0] + p_ref[1]


def kernel(x, edge_index, edge_attr, W, b):
    src = edge_index[0].astype(jnp.int32)
    dst = edge_index[1].astype(jnp.int32)
    We = W[:, :D_EDGE]
    Wx = W[:, D_EDGE:]
    b2 = b.reshape(1, D_OUT)
    zeros = jnp.zeros((N_PAD, D_OUT), jnp.float32)

    g = pl.pallas_call(
        _g_body,
        out_shape=jax.ShapeDtypeStruct((N_NODES, D_NODE), jnp.float32),
        grid=(5,),
        in_specs=[
            pl.BlockSpec((N_NODES // 5, D_NODE), lambda i: (i, 0)),
            pl.BlockSpec((D_OUT, D_NODE), lambda i: (0, 0)),
        ],
        out_specs=pl.BlockSpec((N_NODES // 5, D_NODE), lambda i: (i, 0)),
    )(x, Wx)

    BLK_E = 6400
    a = pl.pallas_call(
        _a_body,
        out_shape=jax.ShapeDtypeStruct((N_EDGES, D_OUT), jnp.float32),
        grid=(N_EDGES // BLK_E,),
        in_specs=[
            pl.BlockSpec((BLK_E, D_EDGE), lambda i: (i, 0)),
            pl.BlockSpec((D_OUT, D_EDGE), lambda i: (0, 0)),
            pl.BlockSpec((1, D_OUT), lambda i: (0, 0)),
        ],
        out_specs=pl.BlockSpec((BLK_E, D_OUT), lambda i: (i, 0)),
    )(edge_attr, We, b2)

    sc_call = pl.kernel(
        _sc_body,
        out_type=jax.ShapeDtypeStruct((NC, N_PAD, D_OUT), jnp.float32),
        mesh=plsc.VectorSubcoreMesh(core_axis_name="c", subcore_axis_name="s"),
        scratch_types=(
            [pltpu.VMEM((CH,), jnp.int32) for _ in range(2 * NBUF)]
            + [pltpu.VMEM((CH, D_OUT), jnp.float32) for _ in range(2 * NBUF)]
            + [pltpu.VMEM_SHARED((N_PAD, D_OUT), jnp.float32)]
            + [pltpu.SemaphoreType.DMA for _ in range(5 * NBUF)]
        ),
    )
    partials = sc_call(g, src, dst, a, zeros)

    out = pl.pallas_call(
        _combine_body,
        out_shape=jax.ShapeDtypeStruct((N_PAD, D_OUT), jnp.float32),
        grid=(8,),
        in_specs=[pl.BlockSpec((NC, N_PAD // 8, D_OUT), lambda i: (0, i, 0))],
        out_specs=pl.BlockSpec((N_PAD // 8, D_OUT), lambda i: (i, 0)),
    )(partials)
    return out[:N_NODES]


# speedup vs baseline: 1.4421x; 1.0250x over previous
import functools

import jax
import jax.numpy as jnp
from jax import lax
from jax.experimental import pallas as pl
from jax.experimental.pallas import tpu as pltpu
import jax.experimental.pallas.tpu_sc as plsc

N_NODES = 10000
N_EDGES = 320000
D_NODE = 128
D_EDGE = 16
D_OUT = 128

NC = 2
NS = 16
NW = NC * NS
EPW = N_EDGES // NW
CH = 40
NCHUNK = EPW // CH
N_PAD = 10240
ROWS_PT = N_PAD // NS
LANES = 16
GRP = D_OUT // LANES
D_HALF = D_OUT // 2


def _pack_pairs(v):
    lo = lax.bitcast_convert_type(
        v[:, :D_HALF].astype(jnp.bfloat16), jnp.uint16).astype(jnp.uint32)
    hi = lax.bitcast_convert_type(
        v[:, D_HALF:].astype(jnp.bfloat16), jnp.uint16).astype(jnp.uint32)
    return lax.bitcast_convert_type(lo | (hi << 16), jnp.int32)


def _g_body(x_ref, wx_ref, g_ref):
    g_ref[...] = lax.dot_general(
        x_ref[...], wx_ref[...], (((1,), (1,)), ((), ())),
        preferred_element_type=jnp.float32)


def _a_body(ea_ref, we_ref, b_ref, a_ref):
    a = lax.dot_general(
        ea_ref[...], we_ref[...], (((1,), (1,)), ((), ())),
        preferred_element_type=jnp.float32) + b_ref[...]
    a_ref[...] = _pack_pairs(a)


NBUF = 3


def _sc_body(g_hbm, src_hbm, dst_hbm, a_hbm, zero_hbm, out_hbm,
             sidx0, sidx1, sidx2, didx0, didx1, didx2,
             rows0, rows1, ain0, ain1, ain2, mout0, mout1, acc,
             sg0, sg1, sa0, sa1, sa2,
             sis0, sis1, sis2, sid0, sid1, sid2, ss0, ss1):
    sidx = (sidx0, sidx1, sidx2)
    didx = (didx0, didx1, didx2)
    rows = (rows0, rows1)
    ain = (ain0, ain1, ain2)
    mout = (mout0, mout1)
    sg = (sg0, sg1)
    sa = (sa0, sa1, sa2)
    sis = (sis0, sis1, sis2)
    sdi = (sid0, sid1, sid2)
    ss = (ss0, ss1)

    cid = lax.axis_index("c")
    sid_ax = lax.axis_index("s")
    wid = sid_ax * NC + cid
    ebase = wid * EPW

    def issue_idx_feeds(c, b3):
        off = ebase + c * CH
        pltpu.async_copy(src_hbm.at[pl.ds(off, CH)], sidx[b3], sis[b3])
        pltpu.async_copy(dst_hbm.at[pl.ds(off, CH)], didx[b3], sdi[b3])
        pltpu.async_copy(a_hbm.at[pl.ds(off, CH)], ain[b3], sa[b3])

    def issue_gather(b3, b2):
        pltpu.make_async_copy(
            src_hbm.at[pl.ds(0, CH)], sidx[b3], sis[b3]).wait()
        pltpu.async_copy(g_hbm.at[sidx[b3]], rows[b2], sg[b2])

    def wait_feeds(b3, b2):
        pltpu.make_async_copy(dst_hbm.at[pl.ds(0, CH)], didx[b3], sdi[b3]).wait()
        pltpu.make_async_copy(a_hbm.at[pl.ds(0, CH)], ain[b3], sa[b3]).wait()
        pltpu.make_async_copy(g_hbm.at[sidx[b3]], rows[b2], sg[b2]).wait()

    def compute_scatter(b3, b2):
        mask = jnp.int32(-65536)

        @plsc.parallel_loop(0, CH, 1, unroll=4)
        def edge_body(e):
            for k in range(D_HALF // LANES):
                s = pl.ds(k * LANES, LANES)
                wa = ain[b3][e, s]
                alo = lax.bitcast_convert_type(wa << 16, jnp.float32)
                ahi = lax.bitcast_convert_type(wa & mask, jnp.float32)
                sh = pl.ds(D_HALF + k * LANES, LANES)
                mout[b2][e, s] = jnp.maximum(rows[b2][e, s] + alo, 0.0)
                mout[b2][e, sh] = jnp.maximum(rows[b2][e, sh] + ahi, 0.0)

        pltpu.async_copy(mout[b2], acc.at[didx[b3]], ss[b2], add=True)

    def wait_scatter(b3, b2):
        pltpu.make_async_copy(mout[b2], acc.at[didx[b3]], ss[b2]).wait()

    issue_idx_feeds(0, 0)
    issue_idx_feeds(1, 1)
    issue_gather(0, 0)
    pltpu.sync_copy(zero_hbm.at[pl.ds(sid_ax * ROWS_PT, ROWS_PT)],
                    acc.at[pl.ds(sid_ax * ROWS_PT, ROWS_PT)])
    plsc.subcore_barrier()

    wait_feeds(0, 0)
    compute_scatter(0, 0)
    issue_idx_feeds(2, 2)
    issue_gather(1, 1)

    wait_feeds(1, 1)
    compute_scatter(1, 1)
    wait_scatter(0, 0)
    issue_idx_feeds(3, 0)
    issue_gather(2, 0)

    @pl.loop(0, (NCHUNK - 4) // 6)
    def _grp(k):
        for b in range(6):
            c = 6 * k + 2 + b
            b3 = (2 + b) % 3
            b2 = b % 2
            wait_feeds(b3, b2)
            compute_scatter(b3, b2)
            wait_scatter((b3 + 2) % 3, 1 - b2)
            issue_idx_feeds(c + 2, (b3 + 2) % 3)
            issue_gather((b3 + 1) % 3, 1 - b2)

    wait_feeds(2, 0)
    compute_scatter(2, 0)
    wait_scatter(1, 1)
    issue_gather(0, 1)

    wait_feeds(0, 1)
    compute_scatter(0, 1)
    wait_scatter(2, 0)
    wait_scatter(0, 1)

    plsc.subcore_barrier()
    pltpu.sync_copy(acc.at[pl.ds(sid_ax * ROWS_PT, ROWS_PT)],
                    out_hbm.at[cid, pl.ds(sid_ax * ROWS_PT, ROWS_PT)])


def _combine_body(p_ref, o_ref):
    o_ref[...] = p_ref[0] + p_ref[1]


def kernel(x, edge_index, edge_attr, W, b):
    src = edge_index[0].astype(jnp.int32)
    dst = edge_index[1].astype(jnp.int32)
    We = W[:, :D_EDGE]
    Wx = W[:, D_EDGE:]
    b2 = b.reshape(1, D_OUT)
    zeros = jnp.zeros((N_PAD, D_OUT), jnp.float32)

    g = pl.pallas_call(
        _g_body,
        out_shape=jax.ShapeDtypeStruct((N_NODES, D_NODE), jnp.float32),
        grid=(5,),
        in_specs=[
            pl.BlockSpec((N_NODES // 5, D_NODE), lambda i: (i, 0)),
            pl.BlockSpec((D_OUT, D_NODE), lambda i: (0, 0)),
        ],
        out_specs=pl.BlockSpec((N_NODES // 5, D_OUT), lambda i: (i, 0)),
    )(x, Wx)

    BLK_E = 6400
    a = pl.pallas_call(
        _a_body,
        out_shape=jax.ShapeDtypeStruct((N_EDGES, D_HALF), jnp.int32),
        grid=(N_EDGES // BLK_E,),
        in_specs=[
            pl.BlockSpec((BLK_E, D_EDGE), lambda i: (i, 0)),
            pl.BlockSpec((D_OUT, D_EDGE), lambda i: (0, 0)),
            pl.BlockSpec((1, D_OUT), lambda i: (0, 0)),
        ],
        out_specs=pl.BlockSpec((BLK_E, D_HALF), lambda i: (i, 0)),
    )(edge_attr, We, b2)

    sc_call = pl.kernel(
        _sc_body,
        out_type=jax.ShapeDtypeStruct((NC, N_PAD, D_OUT), jnp.float32),
        mesh=plsc.VectorSubcoreMesh(core_axis_name="c", subcore_axis_name="s"),
        scratch_types=(
            [pltpu.VMEM((CH,), jnp.int32) for _ in range(6)]
            + [pltpu.VMEM((CH, D_OUT), jnp.float32) for _ in range(2)]
            + [pltpu.VMEM((CH, D_HALF), jnp.int32) for _ in range(3)]
            + [pltpu.VMEM((CH, D_OUT), jnp.float32) for _ in range(2)]
            + [pltpu.VMEM_SHARED((N_PAD, D_OUT), jnp.float32)]
            + [pltpu.SemaphoreType.DMA for _ in range(13)]
        ),
    )
    partials = sc_call(g, src, dst, a, zeros)

    out = pl.pallas_call(
        _combine_body,
        out_shape=jax.ShapeDtypeStruct((N_PAD, D_OUT), jnp.float32),
        grid=(8,),
        in_specs=[pl.BlockSpec((NC, N_PAD // 8, D_OUT), lambda i: (0, i, 0))],
        out_specs=pl.BlockSpec((N_PAD // 8, D_OUT), lambda i: (i, 0)),
    )(partials)
    return out[:N_NODES]


# speedup vs baseline: 1.8880x; 1.3092x over previous
import functools

import jax
import jax.numpy as jnp
from jax import lax
from jax.experimental import pallas as pl
from jax.experimental.pallas import tpu as pltpu
import jax.experimental.pallas.tpu_sc as plsc

N_NODES = 10000
N_EDGES = 320000
D_NODE = 128
D_EDGE = 16
D_OUT = 128

NC = 2
NS = 16
NW = NC * NS
EPW = N_EDGES // NW
CH = 40
NCHUNK = EPW // CH
N_PAD = 10240
ROWS_PT = N_PAD // NS
LANES = 16
GRP = D_OUT // LANES
D_HALF = D_OUT // 2


def _pack_pairs(v):
    lo = lax.bitcast_convert_type(
        v[:, :D_HALF].astype(jnp.bfloat16), jnp.uint16).astype(jnp.uint32)
    hi = lax.bitcast_convert_type(
        v[:, D_HALF:].astype(jnp.bfloat16), jnp.uint16).astype(jnp.uint32)
    return lax.bitcast_convert_type(lo | (hi << 16), jnp.int32)


def _g_body(x_ref, wx_ref, g_ref):
    g_ref[...] = lax.dot_general(
        x_ref[...], wx_ref[...], (((1,), (1,)), ((), ())),
        preferred_element_type=jnp.float32)


def _a_body(ea_ref, we_ref, b_ref, a_ref):
    a = lax.dot_general(
        ea_ref[...], we_ref[...], (((1,), (1,)), ((), ())),
        preferred_element_type=jnp.float32) + b_ref[...]
    a_ref[...] = _pack_pairs(a)


NBUF = 3


def _sc_body(g_hbm, src_hbm, dst_hbm, a_hbm, zero_hbm, out_hbm,
             sidx0, sidx1, sidx2, didx0, didx1, didx2,
             rows0, rows1, ain0, ain1, ain2, mout0, mout1, acc,
             sg0, sg1, sa0, sa1, sa2,
             sis0, sis1, sis2, sid0, sid1, sid2, ss0, ss1):
    sidx = (sidx0, sidx1, sidx2)
    didx = (didx0, didx1, didx2)
    rows = (rows0, rows1)
    ain = (ain0, ain1, ain2)
    mout = (mout0, mout1)
    sg = (sg0, sg1)
    sa = (sa0, sa1, sa2)
    sis = (sis0, sis1, sis2)
    sdi = (sid0, sid1, sid2)
    ss = (ss0, ss1)

    cid = lax.axis_index("c")
    sid_ax = lax.axis_index("s")
    wid = sid_ax * NC + cid
    ebase = wid * EPW

    def issue_sidx(c, b3):
        off = ebase + c * CH
        pltpu.async_copy(src_hbm.at[pl.ds(off, CH)], sidx[b3], sis[b3])

    def issue_dain(c, b3):
        off = ebase + c * CH
        pltpu.async_copy(dst_hbm.at[pl.ds(off, CH)], didx[b3], sdi[b3])
        pltpu.async_copy(a_hbm.at[pl.ds(off, CH)], ain[b3], sa[b3])

    def issue_gather(b3, b2):
        pltpu.make_async_copy(
            src_hbm.at[pl.ds(0, CH)], sidx[b3], sis[b3]).wait()
        pltpu.async_copy(g_hbm.at[sidx[b3]], rows[b2], sg[b2])

    def wait_feeds(b3, b2):
        pltpu.make_async_copy(dst_hbm.at[pl.ds(0, CH)], didx[b3], sdi[b3]).wait()
        pltpu.make_async_copy(a_hbm.at[pl.ds(0, CH)], ain[b3], sa[b3]).wait()
        pltpu.make_async_copy(g_hbm.at[sidx[b3]], rows[b2], sg[b2]).wait()

    def compute_scatter(b3, b2):
        mask = jnp.int32(-65536)

        @plsc.parallel_loop(0, CH, 1, unroll=4)
        def edge_body(e):
            for k in range(D_HALF // LANES):
                s = pl.ds(k * LANES, LANES)
                wa = ain[b3][e, s]
                alo = lax.bitcast_convert_type(wa << 16, jnp.float32)
                ahi = lax.bitcast_convert_type(wa & mask, jnp.float32)
                sh = pl.ds(D_HALF + k * LANES, LANES)
                mout[b2][e, s] = jnp.maximum(rows[b2][e, s] + alo, 0.0)
                mout[b2][e, sh] = jnp.maximum(rows[b2][e, sh] + ahi, 0.0)

        pltpu.async_copy(mout[b2], acc.at[didx[b3]], ss[b2], add=True)

    def wait_scatter(b3, b2):
        pltpu.make_async_copy(mout[b2], acc.at[didx[b3]], ss[b2]).wait()

    issue_sidx(0, 0)
    issue_sidx(1, 1)
    issue_dain(0, 0)
    issue_dain(1, 1)
    issue_gather(0, 0)
    pltpu.sync_copy(zero_hbm.at[pl.ds(sid_ax * ROWS_PT, ROWS_PT)],
                    acc.at[pl.ds(sid_ax * ROWS_PT, ROWS_PT)])
    plsc.subcore_barrier()

    issue_sidx(2, 2)
    issue_gather(1, 1)
    wait_feeds(0, 0)
    compute_scatter(0, 0)
    issue_dain(2, 2)

    issue_sidx(3, 0)
    issue_gather(2, 0)
    wait_feeds(1, 1)
    compute_scatter(1, 1)
    wait_scatter(0, 0)
    issue_dain(3, 0)

    @pl.loop(0, (NCHUNK - 4) // 6)
    def _grp(k):
        for b in range(6):
            c = 6 * k + 2 + b
            b3 = (2 + b) % 3
            b2 = b % 2
            issue_sidx(c + 2, (b3 + 2) % 3)
            issue_gather((b3 + 1) % 3, 1 - b2)
            wait_feeds(b3, b2)
            compute_scatter(b3, b2)
            wait_scatter((b3 + 2) % 3, 1 - b2)
            issue_dain(c + 2, (b3 + 2) % 3)

    issue_gather(0, 1)
    wait_feeds(2, 0)
    compute_scatter(2, 0)
    wait_scatter(1, 1)

    wait_feeds(0, 1)
    compute_scatter(0, 1)
    wait_scatter(2, 0)
    wait_scatter(0, 1)

    plsc.subcore_barrier()
    pltpu.sync_copy(acc.at[pl.ds(sid_ax * ROWS_PT, ROWS_PT)],
                    out_hbm.at[cid, pl.ds(sid_ax * ROWS_PT, ROWS_PT)])


def _combine_body(p_ref, o_ref):
    o_ref[...] = p_ref[0] + p_ref[1]


def kernel(x, edge_index, edge_attr, W, b):
    src = edge_index[0].astype(jnp.int32)
    dst = edge_index[1].astype(jnp.int32)
    We = W[:, :D_EDGE]
    Wx = W[:, D_EDGE:]
    b2 = b.reshape(1, D_OUT)
    zeros = jnp.zeros((N_PAD, D_OUT), jnp.float32)

    g = pl.pallas_call(
        _g_body,
        out_shape=jax.ShapeDtypeStruct((N_NODES, D_NODE), jnp.float32),
        grid=(5,),
        in_specs=[
            pl.BlockSpec((N_NODES // 5, D_NODE), lambda i: (i, 0)),
            pl.BlockSpec((D_OUT, D_NODE), lambda i: (0, 0)),
        ],
        out_specs=pl.BlockSpec((N_NODES // 5, D_OUT), lambda i: (i, 0)),
    )(x, Wx)

    BLK_E = 6400
    a = pl.pallas_call(
        _a_body,
        out_shape=jax.ShapeDtypeStruct((N_EDGES, D_HALF), jnp.int32),
        grid=(N_EDGES // BLK_E,),
        in_specs=[
            pl.BlockSpec((BLK_E, D_EDGE), lambda i: (i, 0)),
            pl.BlockSpec((D_OUT, D_EDGE), lambda i: (0, 0)),
            pl.BlockSpec((1, D_OUT), lambda i: (0, 0)),
        ],
        out_specs=pl.BlockSpec((BLK_E, D_HALF), lambda i: (i, 0)),
    )(edge_attr, We, b2)

    sc_call = pl.kernel(
        _sc_body,
        out_type=jax.ShapeDtypeStruct((NC, N_PAD, D_OUT), jnp.float32),
        mesh=plsc.VectorSubcoreMesh(core_axis_name="c", subcore_axis_name="s"),
        scratch_types=(
            [pltpu.VMEM((CH,), jnp.int32) for _ in range(6)]
            + [pltpu.VMEM((CH, D_OUT), jnp.float32) for _ in range(2)]
            + [pltpu.VMEM((CH, D_HALF), jnp.int32) for _ in range(3)]
            + [pltpu.VMEM((CH, D_OUT), jnp.float32) for _ in range(2)]
            + [pltpu.VMEM_SHARED((N_PAD, D_OUT), jnp.float32)]
            + [pltpu.SemaphoreType.DMA for _ in range(13)]
        ),
    )
    partials = sc_call(g, src, dst, a, zeros)

    out = pl.pallas_call(
        _combine_body,
        out_shape=jax.ShapeDtypeStruct((N_NODES, D_OUT), jnp.float32),
        grid=(5,),
        in_specs=[pl.BlockSpec((NC, N_NODES // 5, D_OUT), lambda i: (0, i, 0))],
        out_specs=pl.BlockSpec((N_NODES // 5, D_OUT), lambda i: (i, 0)),
    )(partials)
    return out
